# baseline (speedup 1.0000x reference)
import functools

import jax
import jax.numpy as jnp
from jax import lax
from jax.experimental import pallas as pl
from jax.experimental.pallas import tpu as pltpu
from jax.experimental.pallas import tpu_sc as plsc

NC = 2
NS = 16
LANES = 16

CHUNK = 128
KB = 16
SUPER = KB * CHUNK

SA = 3
SB = 7

_SC_PARAMS = pltpu.CompilerParams(needs_layout_passes=False,
                                  use_tc_tiling_on_sc=False)


def _rsqrt_nr(d):
    bits = plsc.bitcast(d, jnp.int32)
    y = plsc.bitcast(jnp.int32(0x5F3759DF) - (bits >> 1), jnp.float32)
    for _ in range(3):
        y = y * (1.5 - 0.5 * d * y * y)
    return jnp.where(d > 0.0, y, 0.0)


def _zero_rows(zbuf, n_rows):
    zero16 = jnp.zeros((LANES,), jnp.float32)

    def zrow(j, carry):
        zbuf[j, :] = zero16
        return carry
    lax.fori_loop(0, n_rows, zrow, None)


def _scale_rows(msgs3, norm2):
    def grp(j, carry):
        for g in range(CHUNK // LANES):
            nm16 = norm2[j, pl.ds(g * LANES, LANES)]
            for i in range(LANES):
                r = g * LANES + i
                msgs3[j, r, :] = msgs3[j, r, :] * nm16[i]
        return carry
    lax.fori_loop(0, KB, grp, None)


def _fire_gathers(src_hbm, row2, msgs3, sem):
    for j in range(KB):
        pltpu.async_copy(src_hbm.at[row2.at[j]], msgs3.at[j], sem)


def _drain_gathers(src_hbm, row2, msgs3, sem):
    for j in range(KB):
        pltpu.make_async_copy(src_hbm.at[row2.at[j]], msgs3.at[j], sem).wait()


def _fire_scatters(msgs3, col2, agg_sh, sem):
    for j in range(KB):
        pltpu.async_copy(msgs3.at[j], agg_sh.at[col2.at[j]], sem, add=True)


def _drain_scatters(msgs3, col2, agg_sh, sem):
    for j in range(KB):
        pltpu.make_async_copy(msgs3.at[j], agg_sh.at[col2.at[j]], sem).wait()


def _edge_pipeline(my_supers, max_supers, rbase, src_hbm,
                   row_hbm, col_hbm, w_hbm,
                   rows, cols, ws, mss, lsem, gsem, ssem, agg_sh):
    def fire_loads(i, h):
        pltpu.async_copy(row_hbm.at[pl.ds(rbase(i), KB), :], rows[h], lsem[h])
        pltpu.async_copy(col_hbm.at[pl.ds(rbase(i), KB), :], cols[h], lsem[h])
        pltpu.async_copy(w_hbm.at[pl.ds(rbase(i), KB), :], ws[h], lsem[h])

    def drain_loads(i, h):
        pltpu.make_async_copy(row_hbm.at[pl.ds(rbase(i), KB), :],
                              rows[h], lsem[h]).wait()
        pltpu.make_async_copy(col_hbm.at[pl.ds(rbase(i), KB), :],
                              cols[h], lsem[h]).wait()
        pltpu.make_async_copy(w_hbm.at[pl.ds(rbase(i), KB), :],
                              ws[h], lsem[h]).wait()

    fire_loads(0, 0)

    def edge_pair(kk, carry):
        for h in range(2):
            i = kk * 2 + h
            oh = 1 - h

            @pl.when(i < my_supers)
            def _():
                drain_loads(i, h)
                _fire_gathers(src_hbm, rows[h], mss[h], gsem[h])

                @pl.when(i > 0)
                def _():
                    _drain_scatters(mss[oh], cols[oh], agg_sh, ssem[oh])

                @pl.when(i + 1 < my_supers)
                def _():
                    fire_loads(i + 1, oh)
                _drain_gathers(src_hbm, rows[h], mss[h], gsem[h])
                _scale_rows(mss[h], ws[h])
                _fire_scatters(mss[h], cols[h], agg_sh, ssem[h])
        return carry
    lax.fori_loop(0, (max_supers + 1) // 2, edge_pair, None)
    _drain_scatters(mss[0], cols[0], agg_sh, ssem[0])


def _make_sc_layer1(N_PAD, E_PAD, F):
    n_slice = N_PAD // NS
    deg_supers = E_PAD // NS // SUPER
    assert NS * (SA + SB) * SUPER == E_PAD
    mesh = plsc.VectorSubcoreMesh(core_axis_name="c", subcore_axis_name="s",
                                  num_cores=NC, num_subcores=NS)

    edge_buf = lambda dt: pltpu.VMEM((KB, CHUNK), dt)

    @functools.partial(
        pl.kernel,
        out_type=(jax.ShapeDtypeStruct((N_PAD, F), jnp.float32),
                  jax.ShapeDtypeStruct((NC * N_PAD, F), jnp.float32),
                  jax.ShapeDtypeStruct((NC, N_PAD, F), jnp.float32)),
        mesh=mesh,
        compiler_params=_SC_PARAMS,
        scratch_types=[
            pltpu.VMEM((n_slice, F), jnp.float32),
            pltpu.VMEM((n_slice, F), jnp.float32),
            pltpu.VMEM((n_slice,), jnp.float32),
            edge_buf(jnp.int32), edge_buf(jnp.int32),
            edge_buf(jnp.int32), edge_buf(jnp.int32),
            edge_buf(jnp.float32), edge_buf(jnp.float32),
            pltpu.VMEM((KB, CHUNK, F), jnp.float32),
            pltpu.VMEM((KB, CHUNK, F), jnp.float32),
            pltpu.VMEM_SHARED((N_PAD,), jnp.float32),
            pltpu.VMEM_SHARED((N_PAD, F), jnp.float32),
        ] + [pltpu.SemaphoreType.DMA] * 6,
    )
    def sc_layer1(out0_hbm, ei_hbm, w_hbm,
                  dis_hbm, t_hbm, agg_hbm,
                  zbuf, dbuf, zvec,
                  rowA, rowB, colA, colB, wA, wB, msA, msB,
                  deg_sh, agg_sh,
                  l0, l1, g0, g1, s0, s1):
        c = lax.axis_index("c")
        s = lax.axis_index("s")
        rows = (rowA, rowB)
        cols = (colA, colB)
        ws = (wA, wB)
        mss = (msA, msB)
        lsem = (l0, l1)
        gsem = (g0, g1)
        ssem = (s0, s1)
        row_hbm = ei_hbm.at[0]
        col_hbm = ei_hbm.at[1]

        _zero_rows(zbuf, n_slice)
        zero16 = jnp.zeros((LANES,), jnp.float32)

        def zv(j, carry):
            zvec[pl.ds(j * LANES, LANES)] = zero16
            return carry
        lax.fori_loop(0, n_slice // LANES, zv, None)

        nbase = s * n_slice
        pltpu.sync_copy(zvec, deg_sh.at[pl.ds(nbase, n_slice)])
        pltpu.sync_copy(zbuf, agg_sh.at[pl.ds(nbase, n_slice), :])
        plsc.subcore_barrier()

        def deg_rbase(i):
            return (s * deg_supers + i) * KB

        def deg_fire_loads(i, h):
            pltpu.async_copy(col_hbm.at[pl.ds(deg_rbase(i), KB), :],
                             cols[h], lsem[h])
            pltpu.async_copy(w_hbm.at[pl.ds(deg_rbase(i), KB), :],
                             ws[h], lsem[h])

        def deg_drain_loads(i, h):
            pltpu.make_async_copy(col_hbm.at[pl.ds(deg_rbase(i), KB), :],
                                  cols[h], lsem[h]).wait()
            pltpu.make_async_copy(w_hbm.at[pl.ds(deg_rbase(i), KB), :],
                                  ws[h], lsem[h]).wait()

        deg_fire_loads(0, 0)

        def deg_pair(kk, carry):
            for h in range(2):
                i = kk * 2 + h
                oh = 1 - h

                @pl.when(i < deg_supers)
                def _():
                    deg_drain_loads(i, h)

                    @pl.when(i > 0)
                    def _():
                        for j in range(KB):
                            pltpu.make_async_copy(
                                ws[oh].at[j], deg_sh.at[cols[oh].at[j]],
                                ssem[oh]).wait()

                    @pl.when(i + 1 < deg_supers)
                    def _():
                        deg_fire_loads(i + 1, oh)
                    for j in range(KB):
                        pltpu.async_copy(ws[h].at[j], deg_sh.at[cols[h].at[j]],
                                         ssem[h], add=True)
            return carry
        lax.fori_loop(0, (deg_supers + 1) // 2, deg_pair, None)
        lastp = (deg_supers - 1) % 2
        for j in range(KB):
            pltpu.make_async_copy(ws[lastp].at[j],
                                  deg_sh.at[cols[lastp].at[j]],
                                  ssem[lastp]).wait()
        plsc.subcore_barrier()

        pltpu.sync_copy(deg_sh.at[pl.ds(nbase, n_slice)], zvec)

        def dis_step(j, carry):
            d = zvec[pl.ds(j * LANES, LANES)]
            zvec[pl.ds(j * LANES, LANES)] = _rsqrt_nr(d)
            return carry
        lax.fori_loop(0, n_slice // LANES, dis_step, None)

        pltpu.sync_copy(out0_hbm.at[pl.ds(nbase, n_slice), :], zbuf)

        def tscale(g, carry):
            nm16 = zvec[pl.ds(g * LANES, LANES)]
            for i in range(LANES):
                r = g * LANES + i
                b = jnp.full((LANES,), 1.0, jnp.float32) * nm16[i]
                zbuf[r, :] = zbuf[r, :] * nm16[i]
                dbuf[r, :] = b
            return carry
        lax.fori_loop(0, n_slice // LANES, tscale, None)
        pltpu.sync_copy(zbuf, t_hbm.at[pl.ds(c * N_PAD + nbase, n_slice), :])

        @pl.when(c == 0)
        def _():
            pltpu.sync_copy(dbuf, dis_hbm.at[pl.ds(nbase, n_slice), :])
        plsc.subcore_barrier()

        my_supers = jnp.where(c == 0, SA, SB)
        off = c * (NS * SA) + s * my_supers

        def rbase(i):
            return (off + i) * KB

        my_t = t_hbm.at[pl.ds(c * N_PAD, N_PAD), :]
        _edge_pipeline(my_supers, max(SA, SB), rbase, my_t,
                       row_hbm, col_hbm, w_hbm,
                       rows, cols, ws, mss, lsem, gsem, ssem, agg_sh)
        plsc.subcore_barrier()

        pltpu.sync_copy(agg_sh.at[pl.ds(nbase, n_slice), :],
                        agg_hbm.at[c, pl.ds(nbase, n_slice), :])

    return sc_layer1


def _make_sc_layer2(N_PAD, E_PAD, F):
    n_slice = N_PAD // NS
    assert NS * (SA + SB) * SUPER == E_PAD
    mesh = plsc.VectorSubcoreMesh(core_axis_name="c", subcore_axis_name="s",
                                  num_cores=NC, num_subcores=NS)

    edge_buf = lambda dt: pltpu.VMEM((KB, CHUNK), dt)

    @functools.partial(
        pl.kernel,
        out_type=jax.ShapeDtypeStruct((NC, N_PAD, F), jnp.float32),
        mesh=mesh,
        compiler_params=_SC_PARAMS,
        scratch_types=[
            pltpu.VMEM((n_slice, F), jnp.float32),
            edge_buf(jnp.int32), edge_buf(jnp.int32),
            edge_buf(jnp.int32), edge_buf(jnp.int32),
            edge_buf(jnp.float32), edge_buf(jnp.float32),
            pltpu.VMEM((KB, CHUNK, F), jnp.float32),
            pltpu.VMEM((KB, CHUNK, F), jnp.float32),
            pltpu.VMEM_SHARED((N_PAD, F), jnp.float32),
        ] + [pltpu.SemaphoreType.DMA] * 6,
    )
    def sc_layer2(hp_hbm, ei_hbm, w_hbm,
                  agg_hbm,
                  zbuf, rowA, rowB, colA, colB, wA, wB, msA, msB,
                  agg_sh, l0, l1, g0, g1, s0, s1):
        c = lax.axis_index("c")
        s = lax.axis_index("s")
        rows = (rowA, rowB)
        cols = (colA, colB)
        ws = (wA, wB)
        mss = (msA, msB)
        lsem = (l0, l1)
        gsem = (g0, g1)
        ssem = (s0, s1)
        row_hbm = ei_hbm.at[0]
        col_hbm = ei_hbm.at[1]

        _zero_rows(zbuf, n_slice)
        nbase = s * n_slice
        pltpu.sync_copy(zbuf, agg_sh.at[pl.ds(nbase, n_slice), :])
        plsc.subcore_barrier()

        my_supers = jnp.where(c == 0, SA, SB)
        off = c * (NS * SA) + s * my_supers

        def rbase(i):
            return (off + i) * KB

        _edge_pipeline(my_supers, max(SA, SB), rbase, hp_hbm,
                       row_hbm, col_hbm, w_hbm,
                       rows, cols, ws, mss, lsem, gsem, ssem, agg_sh)
        plsc.subcore_barrier()

        pltpu.sync_copy(agg_sh.at[pl.ds(nbase, n_slice), :],
                        agg_hbm.at[c, pl.ds(nbase, n_slice), :])

    return sc_layer2


def _tc_first(x, w, b, N_PAD):
    N = x.shape[0]
    H2 = w.shape[1]
    H = H2 // 2

    def body(x_ref, w_ref, b_ref, o0_ref, r_ref):
        y = jnp.dot(x_ref[...], w_ref[...],
                    preferred_element_type=jnp.float32) + b_ref[...]
        o0_ref[0:N, :] = y[:, :H]
        o0_ref[N:N_PAD, :] = jnp.zeros((N_PAD - N, H), jnp.float32)
        r_ref[...] = y[:, H:]
    return pl.pallas_call(
        body,
        out_shape=(jax.ShapeDtypeStruct((N_PAD, H), jnp.float32),
                   jax.ShapeDtypeStruct((N, H), jnp.float32)),
    )(x, w, b)


def _tc_mid(agg, root, dis16):
    N = root.shape[0]
    N_PAD, F = dis16.shape

    def body(a_ref, r_ref, d_ref, h_ref, hp_ref):
        d = d_ref[0:N, :]
        h = jnp.maximum(d * (a_ref[0, 0:N, :] + a_ref[1, 0:N, :]) + r_ref[...],
                        0.0)
        h_ref[...] = h
        hp_ref[0:N, :] = d * h
        hp_ref[N:N_PAD, :] = jnp.zeros((N_PAD - N, F), jnp.float32)
    return pl.pallas_call(
        body,
        out_shape=(jax.ShapeDtypeStruct((N, F), jnp.float32),
                   jax.ShapeDtypeStruct((N_PAD, F), jnp.float32)),
    )(agg, root, dis16)


def _tc_final(g2, dis16, h, w2cat, b2):
    N = h.shape[0]

    def body(g_ref, d_ref, h_ref, w_ref, b_ref, o_ref):
        ag = d_ref[0:N, :] * (g_ref[0, 0:N, :] + g_ref[1, 0:N, :])
        z = jnp.dot(jnp.concatenate([ag, h_ref[...]], axis=1), w_ref[...],
                    preferred_element_type=jnp.float32) + b_ref[...]
        z = jnp.maximum(z, 0.0)
        m = jnp.max(z, axis=-1, keepdims=True)
        e = jnp.exp(z - m)
        lse = jnp.log(jnp.sum(e, axis=-1, keepdims=True)) + m
        o_ref[...] = z - lse
    return pl.pallas_call(
        body,
        out_shape=jax.ShapeDtypeStruct((N, w2cat.shape[1]), jnp.float32),
    )(g2, dis16, h, w2cat, b2)


def kernel(x, edge_index, edge_weight, iw1, rw1, b1, iw2, rw2, b2):
    N, F_in = x.shape
    E = edge_index.shape[1]
    H = iw1.shape[2]
    C = iw2.shape[2]

    n_tiles = NC * NS
    N_PAD = ((N + n_tiles * LANES - 1) // (n_tiles * LANES)) * (n_tiles * LANES)
    e_gran = NS * (SA + SB) * SUPER
    E_PAD = ((E + e_gran - 1) // e_gran) * e_gran
    E2 = E_PAD // CHUNK

    pad_e = E_PAD - E
    ei_p = jnp.pad(edge_index, ((0, 0), (0, pad_e))).reshape(2, E2, CHUNK)
    w_p = jnp.pad(edge_weight, (0, pad_e)).reshape(E2, CHUNK)

    w1cat = jnp.concatenate([iw1[0], rw1[0, 0]], axis=1)
    b1cat = jnp.concatenate([jnp.zeros((H,), jnp.float32),
                             b1.reshape(H)]).reshape(1, 2 * H)
    out0_pad, root1 = _tc_first(x, w1cat, b1cat, N_PAD)

    sc1 = _make_sc_layer1(N_PAD, E_PAD, H)
    dis16, _, agg1 = sc1(out0_pad, ei_p, w_p)

    h, hp_pad = _tc_mid(agg1, root1, dis16)

    sc2 = _make_sc_layer2(N_PAD, E_PAD, C)
    g2 = sc2(hp_pad, ei_p, w_p)

    w2cat = jnp.concatenate([iw2[0], rw2[0, 0]], axis=0)
    b2r = b2.reshape(1, C)
    return _tc_final(g2, dis16, h, w2cat, b2r)


# speedup vs baseline: 1.1752x; 1.1752x over previous
import functools

import jax
import jax.numpy as jnp
from jax import lax
from jax.experimental import pallas as pl
from jax.experimental.pallas import tpu as pltpu
from jax.experimental.pallas import tpu_sc as plsc

NC = 2
NS = 16
LANES = 16

CHUNK = 128
KB = 16
SUPER = KB * CHUNK

SA = 7
SB = 3

_SC_PARAMS = pltpu.CompilerParams(needs_layout_passes=False,
                                  use_tc_tiling_on_sc=False)


def _rsqrt_nr(d):
    bits = plsc.bitcast(d, jnp.int32)
    y = plsc.bitcast(jnp.int32(0x5F3759DF) - (bits >> 1), jnp.float32)
    for _ in range(3):
        y = y * (1.5 - 0.5 * d * y * y)
    return jnp.where(d > 0.0, y, 0.0)


def _zero_rows(zbuf, n_rows):
    zero16 = jnp.zeros((LANES,), jnp.float32)

    def zrow(j, carry):
        zbuf[j, :] = zero16
        return carry
    lax.fori_loop(0, n_rows, zrow, None)


def _scale_rows(msgs3, norm2):
    def grp(j, carry):
        for g in range(CHUNK // LANES):
            nm16 = norm2[j, pl.ds(g * LANES, LANES)]
            for i in range(LANES):
                r = g * LANES + i
                msgs3[j, r, :] = msgs3[j, r, :] * nm16[i]
        return carry
    lax.fori_loop(0, KB, grp, None)


def _fire_gathers(src_hbm, row2, msgs3, sem):
    for j in range(KB):
        pltpu.async_copy(src_hbm.at[row2.at[j]], msgs3.at[j], sem)


def _drain_gathers(src_hbm, row2, msgs3, sem):
    for j in range(KB):
        pltpu.make_async_copy(src_hbm.at[row2.at[j]], msgs3.at[j], sem).wait()


def _fire_scatters(msgs3, col2, agg_sh, sem):
    for j in range(KB):
        pltpu.async_copy(msgs3.at[j], agg_sh.at[col2.at[j]], sem, add=True)


def _drain_scatters(msgs3, col2, agg_sh, sem):
    for j in range(KB):
        pltpu.make_async_copy(msgs3.at[j], agg_sh.at[col2.at[j]], sem).wait()


def _edge_pipeline(my_supers, max_supers, rbase, src_hbm,
                   row_hbm, col_hbm, w_hbm,
                   rows, cols, ws, mss, lsem, gsem, ssem, agg_sh):
    def fire_loads(i, h):
        pltpu.async_copy(row_hbm.at[pl.ds(rbase(i), KB), :], rows[h], lsem[h])
        pltpu.async_copy(col_hbm.at[pl.ds(rbase(i), KB), :], cols[h], lsem[h])
        pltpu.async_copy(w_hbm.at[pl.ds(rbase(i), KB), :], ws[h], lsem[h])

    def drain_loads(i, h):
        pltpu.make_async_copy(row_hbm.at[pl.ds(rbase(i), KB), :],
                              rows[h], lsem[h]).wait()
        pltpu.make_async_copy(col_hbm.at[pl.ds(rbase(i), KB), :],
                              cols[h], lsem[h]).wait()
        pltpu.make_async_copy(w_hbm.at[pl.ds(rbase(i), KB), :],
                              ws[h], lsem[h]).wait()

    fire_loads(0, 0)
    drain_loads(0, 0)
    _fire_gathers(src_hbm, rows[0], mss[0], gsem[0])

    def edge_pair(kk, carry):
        for h in range(2):
            i = kk * 2 + h
            oh = 1 - h

            @pl.when(i < my_supers)
            def _():
                @pl.when(i > 0)
                def _():
                    _drain_scatters(mss[oh], cols[oh], agg_sh, ssem[oh])

                @pl.when(i + 1 < my_supers)
                def _():
                    fire_loads(i + 1, oh)
                    drain_loads(i + 1, oh)
                    _fire_gathers(src_hbm, rows[oh], mss[oh], gsem[oh])
                _drain_gathers(src_hbm, rows[h], mss[h], gsem[h])
                _scale_rows(mss[h], ws[h])
                _fire_scatters(mss[h], cols[h], agg_sh, ssem[h])
        return carry
    lax.fori_loop(0, (max_supers + 1) // 2, edge_pair, None)
    _drain_scatters(mss[0], cols[0], agg_sh, ssem[0])


def _make_sc_layer1(N_PAD, E_PAD, F):
    n_slice = N_PAD // NS
    deg_supers = E_PAD // NS // SUPER
    assert NS * (SA + SB) * SUPER == E_PAD
    mesh = plsc.VectorSubcoreMesh(core_axis_name="c", subcore_axis_name="s",
                                  num_cores=NC, num_subcores=NS)

    edge_buf = lambda dt: pltpu.VMEM((KB, CHUNK), dt)

    @functools.partial(
        pl.kernel,
        out_type=(jax.ShapeDtypeStruct((N_PAD, F), jnp.float32),
                  jax.ShapeDtypeStruct((NC * N_PAD, F), jnp.float32),
                  jax.ShapeDtypeStruct((NC, N_PAD, F), jnp.float32)),
        mesh=mesh,
        compiler_params=_SC_PARAMS,
        scratch_types=[
            pltpu.VMEM((n_slice, F), jnp.float32),
            pltpu.VMEM((n_slice, F), jnp.float32),
            pltpu.VMEM((n_slice,), jnp.float32),
            edge_buf(jnp.int32), edge_buf(jnp.int32),
            edge_buf(jnp.int32), edge_buf(jnp.int32),
            edge_buf(jnp.float32), edge_buf(jnp.float32),
            pltpu.VMEM((KB, CHUNK, F), jnp.float32),
            pltpu.VMEM((KB, CHUNK, F), jnp.float32),
            pltpu.VMEM_SHARED((N_PAD,), jnp.float32),
            pltpu.VMEM_SHARED((N_PAD, F), jnp.float32),
        ] + [pltpu.SemaphoreType.DMA] * 6,
    )
    def sc_layer1(out0_hbm, ei_hbm, w_hbm,
                  dis_hbm, t_hbm, agg_hbm,
                  zbuf, dbuf, zvec,
                  rowA, rowB, colA, colB, wA, wB, msA, msB,
                  deg_sh, agg_sh,
                  l0, l1, g0, g1, s0, s1):
        c = lax.axis_index("c")
        s = lax.axis_index("s")
        rows = (rowA, rowB)
        cols = (colA, colB)
        ws = (wA, wB)
        mss = (msA, msB)
        lsem = (l0, l1)
        gsem = (g0, g1)
        ssem = (s0, s1)
        row_hbm = ei_hbm.at[0]
        col_hbm = ei_hbm.at[1]

        _zero_rows(zbuf, n_slice)
        zero16 = jnp.zeros((LANES,), jnp.float32)

        def zv(j, carry):
            zvec[pl.ds(j * LANES, LANES)] = zero16
            return carry
        lax.fori_loop(0, n_slice // LANES, zv, None)

        nbase = s * n_slice
        pltpu.sync_copy(zvec, deg_sh.at[pl.ds(nbase, n_slice)])
        pltpu.sync_copy(zbuf, agg_sh.at[pl.ds(nbase, n_slice), :])
        plsc.subcore_barrier()

        def deg_rbase(i):
            return (s * deg_supers + i) * KB

        def deg_fire_loads(i, h):
            pltpu.async_copy(col_hbm.at[pl.ds(deg_rbase(i), KB), :],
                             cols[h], lsem[h])
            pltpu.async_copy(w_hbm.at[pl.ds(deg_rbase(i), KB), :],
                             ws[h], lsem[h])

        def deg_drain_loads(i, h):
            pltpu.make_async_copy(col_hbm.at[pl.ds(deg_rbase(i), KB), :],
                                  cols[h], lsem[h]).wait()
            pltpu.make_async_copy(w_hbm.at[pl.ds(deg_rbase(i), KB), :],
                                  ws[h], lsem[h]).wait()

        deg_fire_loads(0, 0)

        def deg_pair(kk, carry):
            for h in range(2):
                i = kk * 2 + h
                oh = 1 - h

                @pl.when(i < deg_supers)
                def _():
                    deg_drain_loads(i, h)

                    @pl.when(i > 0)
                    def _():
                        for j in range(KB):
                            pltpu.make_async_copy(
                                ws[oh].at[j], deg_sh.at[cols[oh].at[j]],
                                ssem[oh]).wait()

                    @pl.when(i + 1 < deg_supers)
                    def _():
                        deg_fire_loads(i + 1, oh)
                    for j in range(KB):
                        pltpu.async_copy(ws[h].at[j], deg_sh.at[cols[h].at[j]],
                                         ssem[h], add=True)
            return carry
        lax.fori_loop(0, (deg_supers + 1) // 2, deg_pair, None)
        lastp = (deg_supers - 1) % 2
        for j in range(KB):
            pltpu.make_async_copy(ws[lastp].at[j],
                                  deg_sh.at[cols[lastp].at[j]],
                                  ssem[lastp]).wait()
        plsc.subcore_barrier()

        pltpu.sync_copy(deg_sh.at[pl.ds(nbase, n_slice)], zvec)

        def dis_step(j, carry):
            d = zvec[pl.ds(j * LANES, LANES)]
            zvec[pl.ds(j * LANES, LANES)] = _rsqrt_nr(d)
            return carry
        lax.fori_loop(0, n_slice // LANES, dis_step, None)

        pltpu.sync_copy(out0_hbm.at[pl.ds(nbase, n_slice), :], zbuf)

        def tscale(g, carry):
            nm16 = zvec[pl.ds(g * LANES, LANES)]
            for i in range(LANES):
                r = g * LANES + i
                b = jnp.full((LANES,), 1.0, jnp.float32) * nm16[i]
                zbuf[r, :] = zbuf[r, :] * nm16[i]
                dbuf[r, :] = b
            return carry
        lax.fori_loop(0, n_slice // LANES, tscale, None)
        pltpu.sync_copy(zbuf, t_hbm.at[pl.ds(c * N_PAD + nbase, n_slice), :])

        @pl.when(c == 0)
        def _():
            pltpu.sync_copy(dbuf, dis_hbm.at[pl.ds(nbase, n_slice), :])
        plsc.subcore_barrier()

        my_supers = jnp.where(c == 0, SA, SB)
        off = c * (NS * SA) + s * my_supers

        def rbase(i):
            return (off + i) * KB

        my_t = t_hbm.at[pl.ds(c * N_PAD, N_PAD), :]
        _edge_pipeline(my_supers, max(SA, SB), rbase, my_t,
                       row_hbm, col_hbm, w_hbm,
                       rows, cols, ws, mss, lsem, gsem, ssem, agg_sh)
        plsc.subcore_barrier()

        pltpu.sync_copy(agg_sh.at[pl.ds(nbase, n_slice), :],
                        agg_hbm.at[c, pl.ds(nbase, n_slice), :])

    return sc_layer1


def _make_sc_layer2(N_PAD, E_PAD, F):
    n_slice = N_PAD // NS
    assert NS * (SA + SB) * SUPER == E_PAD
    mesh = plsc.VectorSubcoreMesh(core_axis_name="c", subcore_axis_name="s",
                                  num_cores=NC, num_subcores=NS)

    edge_buf = lambda dt: pltpu.VMEM((KB, CHUNK), dt)

    @functools.partial(
        pl.kernel,
        out_type=jax.ShapeDtypeStruct((NC, N_PAD, F), jnp.float32),
        mesh=mesh,
        compiler_params=_SC_PARAMS,
        scratch_types=[
            pltpu.VMEM((n_slice, F), jnp.float32),
            edge_buf(jnp.int32), edge_buf(jnp.int32),
            edge_buf(jnp.int32), edge_buf(jnp.int32),
            edge_buf(jnp.float32), edge_buf(jnp.float32),
            pltpu.VMEM((KB, CHUNK, F), jnp.float32),
            pltpu.VMEM((KB, CHUNK, F), jnp.float32),
            pltpu.VMEM_SHARED((N_PAD, F), jnp.float32),
        ] + [pltpu.SemaphoreType.DMA] * 6,
    )
    def sc_layer2(hp_hbm, ei_hbm, w_hbm,
                  agg_hbm,
                  zbuf, rowA, rowB, colA, colB, wA, wB, msA, msB,
                  agg_sh, l0, l1, g0, g1, s0, s1):
        c = lax.axis_index("c")
        s = lax.axis_index("s")
        rows = (rowA, rowB)
        cols = (colA, colB)
        ws = (wA, wB)
        mss = (msA, msB)
        lsem = (l0, l1)
        gsem = (g0, g1)
        ssem = (s0, s1)
        row_hbm = ei_hbm.at[0]
        col_hbm = ei_hbm.at[1]

        _zero_rows(zbuf, n_slice)
        nbase = s * n_slice
        pltpu.sync_copy(zbuf, agg_sh.at[pl.ds(nbase, n_slice), :])
        plsc.subcore_barrier()

        my_supers = jnp.where(c == 0, SA, SB)
        off = c * (NS * SA) + s * my_supers

        def rbase(i):
            return (off + i) * KB

        _edge_pipeline(my_supers, max(SA, SB), rbase, hp_hbm,
                       row_hbm, col_hbm, w_hbm,
                       rows, cols, ws, mss, lsem, gsem, ssem, agg_sh)
        plsc.subcore_barrier()

        pltpu.sync_copy(agg_sh.at[pl.ds(nbase, n_slice), :],
                        agg_hbm.at[c, pl.ds(nbase, n_slice), :])

    return sc_layer2


def _tc_first(x, w, b, N_PAD):
    N = x.shape[0]
    H2 = w.shape[1]
    H = H2 // 2

    def body(x_ref, w_ref, b_ref, o0_ref, r_ref):
        y = jnp.dot(x_ref[...], w_ref[...],
                    preferred_element_type=jnp.float32) + b_ref[...]
        o0_ref[0:N, :] = y[:, :H]
        o0_ref[N:N_PAD, :] = jnp.zeros((N_PAD - N, H), jnp.float32)
        r_ref[...] = y[:, H:]
    return pl.pallas_call(
        body,
        out_shape=(jax.ShapeDtypeStruct((N_PAD, H), jnp.float32),
                   jax.ShapeDtypeStruct((N, H), jnp.float32)),
    )(x, w, b)


def _tc_mid(agg, root, dis16):
    N = root.shape[0]
    N_PAD, F = dis16.shape

    def body(a_ref, r_ref, d_ref, h_ref, hp_ref):
        d = d_ref[0:N, :]
        h = jnp.maximum(d * (a_ref[0, 0:N, :] + a_ref[1, 0:N, :]) + r_ref[...],
                        0.0)
        h_ref[...] = h
        hp_ref[0:N, :] = d * h
        hp_ref[N:N_PAD, :] = jnp.zeros((N_PAD - N, F), jnp.float32)
    return pl.pallas_call(
        body,
        out_shape=(jax.ShapeDtypeStruct((N, F), jnp.float32),
                   jax.ShapeDtypeStruct((N_PAD, F), jnp.float32)),
    )(agg, root, dis16)


def _tc_final(g2, dis16, h, w2cat, b2):
    N = h.shape[0]

    def body(g_ref, d_ref, h_ref, w_ref, b_ref, o_ref):
        ag = d_ref[0:N, :] * (g_ref[0, 0:N, :] + g_ref[1, 0:N, :])
        z = jnp.dot(jnp.concatenate([ag, h_ref[...]], axis=1), w_ref[...],
                    preferred_element_type=jnp.float32) + b_ref[...]
        z = jnp.maximum(z, 0.0)
        m = jnp.max(z, axis=-1, keepdims=True)
        e = jnp.exp(z - m)
        lse = jnp.log(jnp.sum(e, axis=-1, keepdims=True)) + m
        o_ref[...] = z - lse
    return pl.pallas_call(
        body,
        out_shape=jax.ShapeDtypeStruct((N, w2cat.shape[1]), jnp.float32),
    )(g2, dis16, h, w2cat, b2)


def kernel(x, edge_index, edge_weight, iw1, rw1, b1, iw2, rw2, b2):
    N, F_in = x.shape
    E = edge_index.shape[1]
    H = iw1.shape[2]
    C = iw2.shape[2]

    n_tiles = NC * NS
    N_PAD = ((N + n_tiles * LANES - 1) // (n_tiles * LANES)) * (n_tiles * LANES)
    e_gran = NS * (SA + SB) * SUPER
    E_PAD = ((E + e_gran - 1) // e_gran) * e_gran
    E2 = E_PAD // CHUNK

    pad_e = E_PAD - E
    ei_p = jnp.pad(edge_index, ((0, 0), (0, pad_e))).reshape(2, E2, CHUNK)
    w_p = jnp.pad(edge_weight, (0, pad_e)).reshape(E2, CHUNK)

    w1cat = jnp.concatenate([iw1[0], rw1[0, 0]], axis=1)
    b1cat = jnp.concatenate([jnp.zeros((H,), jnp.float32),
                             b1.reshape(H)]).reshape(1, 2 * H)
    out0_pad, root1 = _tc_first(x, w1cat, b1cat, N_PAD)

    sc1 = _make_sc_layer1(N_PAD, E_PAD, H)
    dis16, _, agg1 = sc1(out0_pad, ei_p, w_p)

    h, hp_pad = _tc_mid(agg1, root1, dis16)

    sc2 = _make_sc_layer2(N_PAD, E_PAD, C)
    g2 = sc2(hp_pad, ei_p, w_p)

    w2cat = jnp.concatenate([iw2[0], rw2[0, 0]], axis=0)
    b2r = b2.reshape(1, C)
    return _tc_final(g2, dis16, h, w2cat, b2r)


# speedup vs baseline: 1.2018x; 1.0226x over previous
import functools

import jax
import jax.numpy as jnp
from jax import lax
from jax.experimental import pallas as pl
from jax.experimental.pallas import tpu as pltpu
from jax.experimental.pallas import tpu_sc as plsc

NC = 2
NS = 16
LANES = 16

CHUNK = 128
KB = 8
SUPER = KB * CHUNK

SA = 13
SB = 7

_SC_PARAMS = pltpu.CompilerParams(needs_layout_passes=False,
                                  use_tc_tiling_on_sc=False)


def _rsqrt_nr(d):
    bits = plsc.bitcast(d, jnp.int32)
    y = plsc.bitcast(jnp.int32(0x5F3759DF) - (bits >> 1), jnp.float32)
    for _ in range(3):
        y = y * (1.5 - 0.5 * d * y * y)
    return jnp.where(d > 0.0, y, 0.0)


def _zero_rows(zbuf, n_rows):
    zero16 = jnp.zeros((LANES,), jnp.float32)

    def zrow(j, carry):
        zbuf[j, :] = zero16
        return carry
    lax.fori_loop(0, n_rows, zrow, None)


def _scale_rows(msgs3, norm2):
    def grp(j, carry):
        for g in range(CHUNK // LANES):
            nm16 = norm2[j, pl.ds(g * LANES, LANES)]
            for i in range(LANES):
                r = g * LANES + i
                msgs3[j, r, :] = msgs3[j, r, :] * nm16[i]
        return carry
    lax.fori_loop(0, KB, grp, None)


def _fire_gathers(src_hbm, row2, msgs3, sem):
    for j in range(KB):
        pltpu.async_copy(src_hbm.at[row2.at[j]], msgs3.at[j], sem)


def _drain_gathers(src_hbm, row2, msgs3, sem):
    for j in range(KB):
        pltpu.make_async_copy(src_hbm.at[row2.at[j]], msgs3.at[j], sem).wait()


def _fire_scatters(msgs3, col2, agg_sh, sem):
    for j in range(KB):
        pltpu.async_copy(msgs3.at[j], agg_sh.at[col2.at[j]], sem, add=True)


def _drain_scatters(msgs3, col2, agg_sh, sem):
    for j in range(KB):
        pltpu.make_async_copy(msgs3.at[j], agg_sh.at[col2.at[j]], sem).wait()


def _edge_pipeline(my_supers, max_supers, rbase, src_hbm,
                   row_hbm, col_hbm, w_hbm,
                   rows, cols, ws, mss, lsem, gsem, ssem, agg_sh):
    def fire_loads(i, h):
        pltpu.async_copy(row_hbm.at[pl.ds(rbase(i), KB), :], rows[h], lsem[h])
        pltpu.async_copy(col_hbm.at[pl.ds(rbase(i), KB), :], cols[h], lsem[h])
        pltpu.async_copy(w_hbm.at[pl.ds(rbase(i), KB), :], ws[h], lsem[h])

    def drain_loads(i, h):
        pltpu.make_async_copy(row_hbm.at[pl.ds(rbase(i), KB), :],
                              rows[h], lsem[h]).wait()
        pltpu.make_async_copy(col_hbm.at[pl.ds(rbase(i), KB), :],
                              cols[h], lsem[h]).wait()
        pltpu.make_async_copy(w_hbm.at[pl.ds(rbase(i), KB), :],
                              ws[h], lsem[h]).wait()

    fire_loads(0, 0)
    drain_loads(0, 0)
    _fire_gathers(src_hbm, rows[0], mss[0], gsem[0])

    def edge_pair(kk, carry):
        for h in range(2):
            i = kk * 2 + h
            oh = 1 - h

            @pl.when(i < my_supers)
            def _():
                @pl.when(i > 0)
                def _():
                    _drain_scatters(mss[oh], cols[oh], agg_sh, ssem[oh])

                @pl.when(i + 1 < my_supers)
                def _():
                    fire_loads(i + 1, oh)
                    drain_loads(i + 1, oh)
                    _fire_gathers(src_hbm, rows[oh], mss[oh], gsem[oh])
                _drain_gathers(src_hbm, rows[h], mss[h], gsem[h])
                _scale_rows(mss[h], ws[h])
                _fire_scatters(mss[h], cols[h], agg_sh, ssem[h])
        return carry
    lax.fori_loop(0, (max_supers + 1) // 2, edge_pair, None)
    _drain_scatters(mss[0], cols[0], agg_sh, ssem[0])


def _make_sc_layer1(N_PAD, E_PAD, F):
    n_slice = N_PAD // NS
    deg_supers = E_PAD // NS // SUPER
    assert NS * (SA + SB) * SUPER == E_PAD
    mesh = plsc.VectorSubcoreMesh(core_axis_name="c", subcore_axis_name="s",
                                  num_cores=NC, num_subcores=NS)

    edge_buf = lambda dt: pltpu.VMEM((KB, CHUNK), dt)

    @functools.partial(
        pl.kernel,
        out_type=(jax.ShapeDtypeStruct((N_PAD, F), jnp.float32),
                  jax.ShapeDtypeStruct((NC * N_PAD, F), jnp.float32),
                  jax.ShapeDtypeStruct((NC, N_PAD, F), jnp.float32)),
        mesh=mesh,
        compiler_params=_SC_PARAMS,
        scratch_types=[
            pltpu.VMEM((n_slice, F), jnp.float32),
            pltpu.VMEM((n_slice, F), jnp.float32),
            pltpu.VMEM((n_slice,), jnp.float32),
            edge_buf(jnp.int32), edge_buf(jnp.int32),
            edge_buf(jnp.int32), edge_buf(jnp.int32),
            edge_buf(jnp.float32), edge_buf(jnp.float32),
            pltpu.VMEM((KB, CHUNK, F), jnp.float32),
            pltpu.VMEM((KB, CHUNK, F), jnp.float32),
            pltpu.VMEM_SHARED((N_PAD,), jnp.float32),
            pltpu.VMEM_SHARED((N_PAD, F), jnp.float32),
        ] + [pltpu.SemaphoreType.DMA] * 6,
    )
    def sc_layer1(out0_hbm, ei_hbm, w_hbm,
                  dis_hbm, t_hbm, agg_hbm,
                  zbuf, dbuf, zvec,
                  rowA, rowB, colA, colB, wA, wB, msA, msB,
                  deg_sh, agg_sh,
                  l0, l1, g0, g1, s0, s1):
        c = lax.axis_index("c")
        s = lax.axis_index("s")
        rows = (rowA, rowB)
        cols = (colA, colB)
        ws = (wA, wB)
        mss = (msA, msB)
        lsem = (l0, l1)
        gsem = (g0, g1)
        ssem = (s0, s1)
        row_hbm = ei_hbm.at[0]
        col_hbm = ei_hbm.at[1]

        _zero_rows(zbuf, n_slice)
        zero16 = jnp.zeros((LANES,), jnp.float32)

        def zv(j, carry):
            zvec[pl.ds(j * LANES, LANES)] = zero16
            return carry
        lax.fori_loop(0, n_slice // LANES, zv, None)

        nbase = s * n_slice
        pltpu.sync_copy(zvec, deg_sh.at[pl.ds(nbase, n_slice)])
        pltpu.sync_copy(zbuf, agg_sh.at[pl.ds(nbase, n_slice), :])
        plsc.subcore_barrier()

        def deg_rbase(i):
            return (s * deg_supers + i) * KB

        def deg_fire_loads(i, h):
            pltpu.async_copy(col_hbm.at[pl.ds(deg_rbase(i), KB), :],
                             cols[h], lsem[h])
            pltpu.async_copy(w_hbm.at[pl.ds(deg_rbase(i), KB), :],
                             ws[h], lsem[h])

        def deg_drain_loads(i, h):
            pltpu.make_async_copy(col_hbm.at[pl.ds(deg_rbase(i), KB), :],
                                  cols[h], lsem[h]).wait()
            pltpu.make_async_copy(w_hbm.at[pl.ds(deg_rbase(i), KB), :],
                                  ws[h], lsem[h]).wait()

        deg_fire_loads(0, 0)

        def deg_pair(kk, carry):
            for h in range(2):
                i = kk * 2 + h
                oh = 1 - h

                @pl.when(i < deg_supers)
                def _():
                    deg_drain_loads(i, h)

                    @pl.when(i > 0)
                    def _():
                        for j in range(KB):
                            pltpu.make_async_copy(
                                ws[oh].at[j], deg_sh.at[cols[oh].at[j]],
                                ssem[oh]).wait()

                    @pl.when(i + 1 < deg_supers)
                    def _():
                        deg_fire_loads(i + 1, oh)
                    for j in range(KB):
                        pltpu.async_copy(ws[h].at[j], deg_sh.at[cols[h].at[j]],
                                         ssem[h], add=True)
            return carry
        lax.fori_loop(0, (deg_supers + 1) // 2, deg_pair, None)
        lastp = (deg_supers - 1) % 2
        for j in range(KB):
            pltpu.make_async_copy(ws[lastp].at[j],
                                  deg_sh.at[cols[lastp].at[j]],
                                  ssem[lastp]).wait()
        plsc.subcore_barrier()

        pltpu.sync_copy(deg_sh.at[pl.ds(nbase, n_slice)], zvec)

        def dis_step(j, carry):
            d = zvec[pl.ds(j * LANES, LANES)]
            zvec[pl.ds(j * LANES, LANES)] = _rsqrt_nr(d)
            return carry
        lax.fori_loop(0, n_slice // LANES, dis_step, None)

        pltpu.sync_copy(out0_hbm.at[pl.ds(nbase, n_slice), :], zbuf)

        def tscale(g, carry):
            nm16 = zvec[pl.ds(g * LANES, LANES)]
            for i in range(LANES):
                r = g * LANES + i
                b = jnp.full((LANES,), 1.0, jnp.float32) * nm16[i]
                zbuf[r, :] = zbuf[r, :] * nm16[i]
                dbuf[r, :] = b
            return carry
        lax.fori_loop(0, n_slice // LANES, tscale, None)
        pltpu.sync_copy(zbuf, t_hbm.at[pl.ds(c * N_PAD + nbase, n_slice), :])

        @pl.when(c == 0)
        def _():
            pltpu.sync_copy(dbuf, dis_hbm.at[pl.ds(nbase, n_slice), :])
        plsc.subcore_barrier()

        my_supers = jnp.where(c == 0, SA, SB)
        off = c * (NS * SA) + s * my_supers

        def rbase(i):
            return (off + i) * KB

        my_t = t_hbm.at[pl.ds(c * N_PAD, N_PAD), :]
        _edge_pipeline(my_supers, max(SA, SB), rbase, my_t,
                       row_hbm, col_hbm, w_hbm,
                       rows, cols, ws, mss, lsem, gsem, ssem, agg_sh)
        plsc.subcore_barrier()

        pltpu.sync_copy(agg_sh.at[pl.ds(nbase, n_slice), :],
                        agg_hbm.at[c, pl.ds(nbase, n_slice), :])

    return sc_layer1


def _make_sc_layer2(N_PAD, E_PAD, F):
    n_slice = N_PAD // NS
    assert NS * (SA + SB) * SUPER == E_PAD
    mesh = plsc.VectorSubcoreMesh(core_axis_name="c", subcore_axis_name="s",
                                  num_cores=NC, num_subcores=NS)

    edge_buf = lambda dt: pltpu.VMEM((KB, CHUNK), dt)

    @functools.partial(
        pl.kernel,
        out_type=jax.ShapeDtypeStruct((NC, N_PAD, F), jnp.float32),
        mesh=mesh,
        compiler_params=_SC_PARAMS,
        scratch_types=[
            pltpu.VMEM((n_slice, F), jnp.float32),
            edge_buf(jnp.int32), edge_buf(jnp.int32),
            edge_buf(jnp.int32), edge_buf(jnp.int32),
            edge_buf(jnp.float32), edge_buf(jnp.float32),
            pltpu.VMEM((KB, CHUNK, F), jnp.float32),
            pltpu.VMEM((KB, CHUNK, F), jnp.float32),
            pltpu.VMEM_SHARED((N_PAD, F), jnp.float32),
        ] + [pltpu.SemaphoreType.DMA] * 6,
    )
    def sc_layer2(hp_hbm, ei_hbm, w_hbm,
                  agg_hbm,
                  zbuf, rowA, rowB, colA, colB, wA, wB, msA, msB,
                  agg_sh, l0, l1, g0, g1, s0, s1):
        c = lax.axis_index("c")
        s = lax.axis_index("s")
        rows = (rowA, rowB)
        cols = (colA, colB)
        ws = (wA, wB)
        mss = (msA, msB)
        lsem = (l0, l1)
        gsem = (g0, g1)
        ssem = (s0, s1)
        row_hbm = ei_hbm.at[0]
        col_hbm = ei_hbm.at[1]

        _zero_rows(zbuf, n_slice)
        nbase = s * n_slice
        pltpu.sync_copy(zbuf, agg_sh.at[pl.ds(nbase, n_slice), :])
        plsc.subcore_barrier()

        my_supers = jnp.where(c == 0, SA, SB)
        off = c * (NS * SA) + s * my_supers

        def rbase(i):
            return (off + i) * KB

        _edge_pipeline(my_supers, max(SA, SB), rbase, hp_hbm,
                       row_hbm, col_hbm, w_hbm,
                       rows, cols, ws, mss, lsem, gsem, ssem, agg_sh)
        plsc.subcore_barrier()

        pltpu.sync_copy(agg_sh.at[pl.ds(nbase, n_slice), :],
                        agg_hbm.at[c, pl.ds(nbase, n_slice), :])

    return sc_layer2


def _tc_first(x, w, b, N_PAD):
    N = x.shape[0]
    H2 = w.shape[1]
    H = H2 // 2

    def body(x_ref, w_ref, b_ref, o0_ref, r_ref):
        y = jnp.dot(x_ref[...], w_ref[...],
                    preferred_element_type=jnp.float32) + b_ref[...]
        o0_ref[0:N, :] = y[:, :H]
        o0_ref[N:N_PAD, :] = jnp.zeros((N_PAD - N, H), jnp.float32)
        r_ref[...] = y[:, H:]
    return pl.pallas_call(
        body,
        out_shape=(jax.ShapeDtypeStruct((N_PAD, H), jnp.float32),
                   jax.ShapeDtypeStruct((N, H), jnp.float32)),
    )(x, w, b)


def _tc_mid(agg, root, dis16):
    N = root.shape[0]
    N_PAD, F = dis16.shape

    def body(a_ref, r_ref, d_ref, h_ref, hp_ref):
        d = d_ref[0:N, :]
        h = jnp.maximum(d * (a_ref[0, 0:N, :] + a_ref[1, 0:N, :]) + r_ref[...],
                        0.0)
        h_ref[...] = h
        hp_ref[0:N, :] = d * h
        hp_ref[N:N_PAD, :] = jnp.zeros((N_PAD - N, F), jnp.float32)
    return pl.pallas_call(
        body,
        out_shape=(jax.ShapeDtypeStruct((N, F), jnp.float32),
                   jax.ShapeDtypeStruct((N_PAD, F), jnp.float32)),
    )(agg, root, dis16)


def _tc_final(g2, dis16, h, w2cat, b2):
    N = h.shape[0]

    def body(g_ref, d_ref, h_ref, w_ref, b_ref, o_ref):
        ag = d_ref[0:N, :] * (g_ref[0, 0:N, :] + g_ref[1, 0:N, :])
        z = jnp.dot(jnp.concatenate([ag, h_ref[...]], axis=1), w_ref[...],
                    preferred_element_type=jnp.float32) + b_ref[...]
        z = jnp.maximum(z, 0.0)
        m = jnp.max(z, axis=-1, keepdims=True)
        e = jnp.exp(z - m)
        lse = jnp.log(jnp.sum(e, axis=-1, keepdims=True)) + m
        o_ref[...] = z - lse
    return pl.pallas_call(
        body,
        out_shape=jax.ShapeDtypeStruct((N, w2cat.shape[1]), jnp.float32),
    )(g2, dis16, h, w2cat, b2)


def kernel(x, edge_index, edge_weight, iw1, rw1, b1, iw2, rw2, b2):
    N, F_in = x.shape
    E = edge_index.shape[1]
    H = iw1.shape[2]
    C = iw2.shape[2]

    n_tiles = NC * NS
    N_PAD = ((N + n_tiles * LANES - 1) // (n_tiles * LANES)) * (n_tiles * LANES)
    e_gran = NS * (SA + SB) * SUPER
    E_PAD = ((E + e_gran - 1) // e_gran) * e_gran
    E2 = E_PAD // CHUNK

    pad_e = E_PAD - E
    ei_p = jnp.pad(edge_index, ((0, 0), (0, pad_e))).reshape(2, E2, CHUNK)
    w_p = jnp.pad(edge_weight, (0, pad_e)).reshape(E2, CHUNK)

    w1cat = jnp.concatenate([iw1[0], rw1[0, 0]], axis=1)
    b1cat = jnp.concatenate([jnp.zeros((H,), jnp.float32),
                             b1.reshape(H)]).reshape(1, 2 * H)
    out0_pad, root1 = _tc_first(x, w1cat, b1cat, N_PAD)

    sc1 = _make_sc_layer1(N_PAD, E_PAD, H)
    dis16, _, agg1 = sc1(out0_pad, ei_p, w_p)

    h, hp_pad = _tc_mid(agg1, root1, dis16)

    sc2 = _make_sc_layer2(N_PAD, E_PAD, C)
    g2 = sc2(hp_pad, ei_p, w_p)

    w2cat = jnp.concatenate([iw2[0], rw2[0, 0]], axis=0)
    b2r = b2.reshape(1, C)
    return _tc_final(g2, dis16, h, w2cat, b2r)
